# baseline (speedup 1.0000x reference)
import jax
import jax.numpy as jnp
from jax import lax
from jax.experimental import pallas as pl
from jax.experimental.pallas import tpu as pltpu

V = 32000
D = 1024
H = 16
L = 2
HID = 2048
E = 8
TOPK = 2
T = 2048
BASE = 10000.0
EPS = 1e-6

BT = 256
TILE = 128
NA = T * TOPK
PAD = NA + E * TILE
NT = PAD // TILE
BV = 1280


def _rnbf16(x):
    u = lax.bitcast_convert_type(x, jnp.uint32)
    r = (u + jnp.uint32(0x7FFF) + ((u >> 16) & jnp.uint32(1))) \
        & jnp.uint32(0xFFFF0000)
    return lax.bitcast_convert_type(r, jnp.float32).astype(jnp.bfloat16)


def _dot(a, b):
    return jnp.dot(_rnbf16(a), _rnbf16(b),
                   preferred_element_type=jnp.float32)


def _dot_nt(a, b):
    return lax.dot_general(_rnbf16(a), _rnbf16(b),
                           (((1,), (1,)), ((), ())),
                           preferred_element_type=jnp.float32)


def _norm_rows(x, w):
    var = jnp.mean(x * x, axis=-1, keepdims=True)
    return x * lax.rsqrt(var + EPS) * w


def _rope(x):
    dh = x.shape[-1]
    half = dh // 2
    inv = 1.0 / (BASE ** (jnp.arange(half, dtype=jnp.float32) / half))
    ang = jnp.arange(x.shape[2], dtype=jnp.float32)[:, None] * inv[None, :]
    cos = jnp.cos(ang)[None, None, :, :]
    sin = jnp.sin(ang)[None, None, :, :]
    x1 = x[..., :half]
    x2 = x[..., half:]
    return jnp.concatenate([x1 * cos - x2 * sin, x1 * sin + x2 * cos], axis=-1)


def _attn_block(x, wq, wk, wv, wo):
    b, t, d = x.shape
    dh = d // H
    q = (x @ wq).reshape(b, t, H, dh).transpose(0, 2, 1, 3)
    k = (x @ wk).reshape(b, t, H, dh).transpose(0, 2, 1, 3)
    v = (x @ wv).reshape(b, t, H, dh).transpose(0, 2, 1, 3)
    q = _rope(q)
    k = _rope(k)
    scores = jnp.einsum("bhqd,bhkd->bhqk", q, k) / jnp.sqrt(float(dh))
    mask = jnp.tril(jnp.ones((t, t), dtype=bool))
    scores = jnp.where(mask[None, None, :, :], scores, -1e9)
    attn = jax.nn.softmax(scores, axis=-1)
    out = jnp.einsum("bhqk,bhkd->bhqd", attn, v)
    out = out.transpose(0, 2, 1, 3).reshape(b, t, d)
    return out @ wo


def _rms_body(x_ref, w_ref, o_ref):
    o_ref[...] = _norm_rows(x_ref[...], w_ref[...])


def _rmsnorm(x, w):
    return pl.pallas_call(
        _rms_body,
        grid=(T // BT,),
        in_specs=[pl.BlockSpec((BT, D), lambda t: (t, 0)),
                  pl.BlockSpec((1, D), lambda t: (0, 0))],
        out_specs=pl.BlockSpec((BT, D), lambda t: (t, 0)),
        out_shape=jax.ShapeDtypeStruct((T, D), jnp.float32),
    )(x, w)


def _router_body(x_ref, rw_ref, e_ref, v_ref):
    logits = _dot(x_ref[...], rw_ref[...])
    col = lax.broadcasted_iota(jnp.int32, (BT, 128), 1)
    neg = jnp.float32(-1e30)
    logits = jnp.where(col < E, logits, neg)
    m1 = jnp.max(logits, axis=-1, keepdims=True)
    e0 = jnp.min(jnp.where(logits == m1, col, 127), axis=-1, keepdims=True)
    l2 = jnp.where(col == e0, neg, logits)
    m2 = jnp.max(l2, axis=-1, keepdims=True)
    e1 = jnp.min(jnp.where(l2 == m2, col, 127), axis=-1, keepdims=True)
    e_ref[...] = jnp.where(col == 0, e0, jnp.where(col == 1, e1, 0))
    v_ref[...] = jnp.where(col == 0, m1, jnp.where(col == 1, m2, 0.0))


def _router(xn, rw_pad):
    return pl.pallas_call(
        _router_body,
        grid=(T // BT,),
        in_specs=[pl.BlockSpec((BT, D), lambda t: (t, 0)),
                  pl.BlockSpec((D, 128), lambda t: (0, 0))],
        out_specs=[pl.BlockSpec((BT, 128), lambda t: (t, 0)),
                   pl.BlockSpec((BT, 128), lambda t: (t, 0))],
        out_shape=[jax.ShapeDtypeStruct((T, 128), jnp.int32),
                   jax.ShapeDtypeStruct((T, 128), jnp.float32)],
    )(xn, rw_pad)


def _ffn1_body(te_ref, xg_ref, w1_ref, w3_ref, z1_ref, z3_ref):
    x = xg_ref[...]
    z1_ref[...] = _dot(x, w1_ref[0])
    z3_ref[...] = _dot(x, w3_ref[0])


def _ffn1(tile_expert, xg, e1, e3):
    grid_spec = pltpu.PrefetchScalarGridSpec(
        num_scalar_prefetch=1,
        grid=(NT,),
        in_specs=[
            pl.BlockSpec((TILE, D), lambda t, te: (t, 0)),
            pl.BlockSpec((1, D, HID), lambda t, te: (te[t], 0, 0)),
            pl.BlockSpec((1, D, HID), lambda t, te: (te[t], 0, 0)),
        ],
        out_specs=[pl.BlockSpec((TILE, HID), lambda t, te: (t, 0)),
                   pl.BlockSpec((TILE, HID), lambda t, te: (t, 0))],
    )
    return pl.pallas_call(
        _ffn1_body,
        grid_spec=grid_spec,
        out_shape=[jax.ShapeDtypeStruct((PAD, HID), jnp.float32)] * 2,
    )(tile_expert, xg, e1, e3)


def _ffn2_body(te_ref, act_ref, w2_ref, wr_ref, y_ref):
    y = _dot(act_ref[...], w2_ref[0])
    y_ref[...] = y * wr_ref[:, :1]


def _ffn2(tile_expert, act, e2, w_rep):
    grid_spec = pltpu.PrefetchScalarGridSpec(
        num_scalar_prefetch=1,
        grid=(NT,),
        in_specs=[
            pl.BlockSpec((TILE, HID), lambda t, te: (t, 0)),
            pl.BlockSpec((1, HID, D), lambda t, te: (te[t], 0, 0)),
            pl.BlockSpec((TILE, 128), lambda t, te: (t, 0)),
        ],
        out_specs=pl.BlockSpec((TILE, D), lambda t, te: (t, 0)),
    )
    return pl.pallas_call(
        _ffn2_body,
        grid_spec=grid_spec,
        out_shape=jax.ShapeDtypeStruct((PAD, D), jnp.float32),
    )(tile_expert, act, e2, w_rep)


def _comb_body(a_ref, b_ref, c_ref, o_ref):
    o_ref[...] = a_ref[...] + (b_ref[...] + c_ref[...])


def _combine(res, y0, y1):
    return pl.pallas_call(
        _comb_body,
        grid=(T // BT,),
        in_specs=[pl.BlockSpec((BT, D), lambda t: (t, 0))] * 3,
        out_specs=pl.BlockSpec((BT, D), lambda t: (t, 0)),
        out_shape=jax.ShapeDtypeStruct((T, D), jnp.float32),
    )(res, y0, y1)


def _lm_body(x_ref, nw_ref, emb_ref, o_ref):
    x = _norm_rows(x_ref[...], nw_ref[...])
    o_ref[...] = _dot_nt(x, emb_ref[...])


def _lm_head(x, final_w, tok_embed):
    BTL = 512
    grid = (V // BV, T // BTL)
    return pl.pallas_call(
        _lm_body,
        grid=grid,
        in_specs=[
            pl.BlockSpec((BTL, D), lambda v, t: (t, 0)),
            pl.BlockSpec((1, D), lambda v, t: (0, 0)),
            pl.BlockSpec((BV, D), lambda v, t: (v, 0)),
        ],
        out_specs=pl.BlockSpec((BTL, BV), lambda v, t: (t, v)),
        out_shape=jax.ShapeDtypeStruct((T, V), jnp.float32),
    )(x, final_w, tok_embed)


def _dispatch(e0, e1, w0, w1):
    flat_e = jnp.concatenate([e0, e1])
    flat_w = jnp.concatenate([w0, w1])
    onehot = jax.nn.one_hot(flat_e, E, dtype=jnp.int32)
    ranks_all = jnp.cumsum(onehot, axis=0)
    rank = jnp.take_along_axis(ranks_all, flat_e[:, None], axis=1)[:, 0] - 1
    counts = ranks_all[-1]
    padded = ((counts + TILE - 1) // TILE) * TILE
    pad_off = jnp.concatenate([jnp.zeros((1,), jnp.int32),
                               jnp.cumsum(padded)[:-1].astype(jnp.int32)])
    slot = pad_off[flat_e] + rank
    gather_idx = jnp.zeros((PAD,), jnp.int32).at[slot].set(
        jnp.arange(NA, dtype=jnp.int32) % T)
    w_pad = jnp.zeros((PAD,), jnp.float32).at[slot].set(flat_w)
    tile_starts = jnp.arange(NT, dtype=jnp.int32) * TILE
    tile_expert = (jnp.searchsorted(pad_off, tile_starts, side="right")
                   .astype(jnp.int32) - 1)
    tile_expert = jnp.clip(tile_expert, 0, E - 1)
    return gather_idx, w_pad, tile_expert, slot[:T], slot[T:]


def kernel(input_ids, tok_embed, attn_norm_w, wq, wk, wv, wo, moe_norm_w,
           router_w, ew1, ew3, ew2, final_norm_w):
    rw_pad = jnp.pad(router_w, ((0, 0), (0, 0), (0, 128 - E)))

    x = jnp.take(tok_embed, input_ids, axis=0)
    for l in range(L):
        xn_a = _norm_rows(x[0], attn_norm_w[l])
        x = x + _attn_block(xn_a[None], wq[l], wk[l], wv[l], wo[l])

        xn = _norm_rows(x[0], moe_norm_w[l])
        e_out, v_out = _router(xn, rw_pad[l])
        e0, e1 = e_out[:, 0], e_out[:, 1]
        w2s = jax.nn.softmax(v_out[:, :2], axis=-1)
        w0, w1 = w2s[:, 0], w2s[:, 1]
        gather_idx, w_pad, tile_expert, slot0, slot1 = _dispatch(e0, e1, w0, w1)
        xg = jnp.take(xn, gather_idx, axis=0)
        w_rep = jnp.broadcast_to(w_pad[:, None], (PAD, 128))
        z1, z3 = _ffn1(tile_expert, xg, ew1[l], ew3[l])
        act = jax.nn.silu(z1) * z3
        yg = _ffn2(tile_expert, act, ew2[l], w_rep)
        y0 = jnp.take(yg, slot0, axis=0)
        y1 = jnp.take(yg, slot1, axis=0)
        x = _combine(x[0], y0, y1)[None]

    logits = _lm_head(x[0], final_norm_w[None, :], tok_embed)
    return logits[None]
